# Initial kernel scaffold; baseline (speedup 1.0000x reference)
import jax
import jax.numpy as jnp
from jax.experimental import pallas as pl


def kernel(img):
    raise NotImplementedError("write your pallas kernel here")


# speedup vs baseline: 5.0233x; 5.0233x over previous
import jax
import jax.numpy as jnp
from jax.experimental import pallas as pl
from jax.experimental.pallas import tpu as pltpu

N, H, W, C = 4, 224, 224, 96
TB = 16


def _upsample_body(prev_ref, mid_ref, next_ref, out_ref):
    rows = mid_ref[0]
    prevrow = prev_ref[0]
    nextrow = next_ref[0]
    rprev = jnp.concatenate([prevrow, rows[:-1]], axis=0)
    rnext = jnp.concatenate([rows[1:], nextrow], axis=0)
    even = 0.25 * rprev + 0.75 * rows
    odd = 0.75 * rows + 0.25 * rnext

    def colup(b):
        bprev = jnp.concatenate([b[:, :1], b[:, :-1]], axis=1)
        bnext = jnp.concatenate([b[:, 1:], b[:, -1:]], axis=1)
        e = 0.25 * bprev + 0.75 * b
        o = 0.75 * b + 0.25 * bnext
        return jnp.stack([e, o], axis=2).reshape(TB, 2 * W, C)

    out_ref[0, :, 0] = colup(even)
    out_ref[0, :, 1] = colup(odd)


def kernel(img):
    nblk = H // TB
    out5 = pl.pallas_call(
        _upsample_body,
        grid=(N, nblk),
        in_specs=[
            pl.BlockSpec((1, 1, W, C),
                         lambda n, t: (n, jnp.maximum(t * TB - 1, 0), 0, 0)),
            pl.BlockSpec((1, TB, W, C), lambda n, t: (n, t, 0, 0)),
            pl.BlockSpec((1, 1, W, C),
                         lambda n, t: (n, jnp.minimum(t * TB + TB, H - 1), 0, 0)),
        ],
        out_specs=pl.BlockSpec((1, TB, 2, 2 * W, C),
                               lambda n, t: (n, t, 0, 0, 0)),
        out_shape=jax.ShapeDtypeStruct((N, H, 2, 2 * W, C), img.dtype),
        compiler_params=pltpu.CompilerParams(
            dimension_semantics=("parallel", "arbitrary")),
    )(img, img, img)
    return out5.reshape(N, 2 * H, 2 * W, C)


# speedup vs baseline: 7.2409x; 1.4415x over previous
import jax
import jax.numpy as jnp
from jax import lax
from jax.experimental import pallas as pl
from jax.experimental.pallas import tpu as pltpu

N, H, W, C = 4, 224, 224, 96
TB = 16
NBLK = H // TB
NSTEPS = N * NBLK


def _upsample_body(prev_ref, mid_ref, next_ref, out_ref, buf_ref, sem_ref):
    n = pl.program_id(0)
    t = pl.program_id(1)
    i = n * NBLK + t
    p = lax.rem(i, 2)

    def dst(a, b):
        return out_ref.at[n, pl.ds(t * TB, TB), a, :, b, :]

    @pl.when(i >= 2)
    def _():
        for q, (a, b) in enumerate(((0, 0), (0, 1), (1, 0), (1, 1))):
            pltpu.make_async_copy(buf_ref.at[p, q], dst(a, b),
                                  sem_ref.at[p]).wait()

    for r in range(TB):
        prow = mid_ref[0, r - 1] if r >= 1 else prev_ref[0, 0]
        crow = mid_ref[0, r]
        nrow = mid_ref[0, r + 1] if r < TB - 1 else next_ref[0, 0]
        for q2, bl in ((0, 0.25 * prow + 0.75 * crow),
                       (2, 0.75 * crow + 0.25 * nrow)):
            sp = jnp.concatenate([bl[:1], bl[:-1]], axis=0)
            sn = jnp.concatenate([bl[1:], bl[-1:]], axis=0)
            buf_ref[p, q2, r] = 0.25 * sp + 0.75 * bl
            buf_ref[p, q2 + 1, r] = 0.75 * bl + 0.25 * sn

    for q, (a, b) in enumerate(((0, 0), (0, 1), (1, 0), (1, 1))):
        pltpu.make_async_copy(buf_ref.at[p, q], dst(a, b),
                              sem_ref.at[p]).start()

    @pl.when(i == NSTEPS - 1)
    def _():
        for pp in (1 - p, p):
            for q, (a, b) in enumerate(((0, 0), (0, 1), (1, 0), (1, 1))):
                pltpu.make_async_copy(buf_ref.at[pp, q], dst(a, b),
                                      sem_ref.at[pp]).wait()


def kernel(img):
    out6 = pl.pallas_call(
        _upsample_body,
        grid=(N, NBLK),
        in_specs=[
            pl.BlockSpec((1, 1, W, C),
                         lambda n, t: (n, jnp.maximum(t * TB - 1, 0), 0, 0)),
            pl.BlockSpec((1, TB, W, C), lambda n, t: (n, t, 0, 0)),
            pl.BlockSpec((1, 1, W, C),
                         lambda n, t: (n, jnp.minimum(t * TB + TB, H - 1), 0, 0)),
        ],
        out_specs=pl.BlockSpec(memory_space=pltpu.MemorySpace.HBM),
        out_shape=jax.ShapeDtypeStruct((N, H, 2, W, 2, C), img.dtype),
        scratch_shapes=[
            pltpu.VMEM((2, 4, TB, W, C), jnp.float32),
            pltpu.SemaphoreType.DMA((2,)),
        ],
        compiler_params=pltpu.CompilerParams(
            dimension_semantics=("parallel", "arbitrary")),
    )(img, img, img)
    return out6.reshape(N, 2 * H, 2 * W, C)
